# baseline (speedup 1.0000x reference)
import functools

import jax
import jax.numpy as jnp
from jax.experimental import pallas as pl
from jax.experimental.pallas import tpu as pltpu
from jax.experimental.pallas import tpu_sc as plsc

N = 2048
E = 131072
NW = 32
ROWS_PER_W = N // NW
PASS_ROWS = ROWS_PER_W // 2
PASS_WORDS = PASS_ROWS * N
ECHUNK = 8192
BLK = 256
NBLK = N // BLK


NCHUNK = E // ECHUNK
UNROLL = 8


def _sc_counts_body(src_hbm, dst_hbm, c_hbm,
                    cblk, sbuf0, dbuf0, sbuf1, dbuf1, sem0, sem1):
    wid = jax.lax.axis_index("s") * 2 + jax.lax.axis_index("c")
    zeros16 = jnp.zeros((16,), jnp.float32)
    ones16 = jnp.ones((16,), jnp.float32)

    def start(c, sbuf, dbuf, sem):
        off = c * ECHUNK
        pltpu.make_async_copy(src_hbm.at[pl.ds(off, ECHUNK)], sbuf, sem).start()
        pltpu.make_async_copy(dst_hbm.at[pl.ds(off, ECHUNK)], dbuf, sem).start()

    def wait(sbuf, dbuf, sem):
        pltpu.make_async_copy(src_hbm.at[pl.ds(0, ECHUNK)], sbuf, sem).wait()
        pltpu.make_async_copy(dst_hbm.at[pl.ds(0, ECHUNK)], dbuf, sem).wait()

    for p in range(2):
        base = wid * ROWS_PER_W + p * PASS_ROWS
        start(0, sbuf0, dbuf0, sem0)

        def zbody(r, _):
            for u in range(N // (16 * UNROLL)):
                for v in range(UNROLL):
                    cblk[r, pl.ds((u * UNROLL + v) * 16, 16)] = zeros16
            return 0

        jax.lax.fori_loop(0, PASS_ROWS, zbody, 0)

        def process(sbuf, dbuf):
            def ibody(j, _):
                for u in range(UNROLL):
                    sl = pl.ds(j * (16 * UNROLL) + u * 16, 16)
                    d = dbuf[sl]
                    s = sbuf[sl]
                    rel = d - base
                    msk = (rel >= 0) & (rel < PASS_ROWS)
                    ridx = jnp.where(msk, rel, 0)
                    plsc.addupdate_scatter(cblk, [ridx, s], ones16, mask=msk)
                return 0

            jax.lax.fori_loop(0, ECHUNK // (16 * UNROLL), ibody, 0)

        def cbody(i, _):
            c = i * 2
            start(c + 1, sbuf1, dbuf1, sem1)
            wait(sbuf0, dbuf0, sem0)
            process(sbuf0, dbuf0)

            @pl.when(c + 2 < NCHUNK)
            def _():
                start(c + 2, sbuf0, dbuf0, sem0)

            wait(sbuf1, dbuf1, sem1)
            process(sbuf1, dbuf1)
            return 0

        jax.lax.fori_loop(0, NCHUNK // 2, cbody, 0)
        pltpu.sync_copy(cblk, c_hbm.at[pl.ds(base, PASS_ROWS), :])


@functools.lru_cache(maxsize=1)
def _sc_counts():
    return pl.kernel(
        _sc_counts_body,
        out_type=jax.ShapeDtypeStruct((N, N), jnp.float32),
        mesh=plsc.VectorSubcoreMesh(
            core_axis_name="c", subcore_axis_name="s",
            num_cores=2, num_subcores=16,
        ),
        scratch_types=[
            pltpu.VMEM((PASS_ROWS, N), jnp.float32),
            pltpu.VMEM((ECHUNK,), jnp.int32),
            pltpu.VMEM((ECHUNK,), jnp.int32),
            pltpu.VMEM((ECHUNK,), jnp.int32),
            pltpu.VMEM((ECHUNK,), jnp.int32),
            pltpu.SemaphoreType.DMA,
            pltpu.SemaphoreType.DMA,
        ],
        compiler_params=pltpu.CompilerParams(needs_layout_passes=False),
    )


def _b16(x):
    return x.astype(jnp.bfloat16)


def _r16(x):
    return x.astype(jnp.bfloat16).astype(jnp.float32)


def _mm(a, b):
    return jax.lax.dot_general(
        a, b, (((1,), (0,)), ((), ())),
        preferred_element_type=jnp.float32,
    )


def _mmT(a, b):
    return jax.lax.dot_general(
        a, b, (((1,), (1,)), ((), ())),
        preferred_element_type=jnp.float32,
    )


def _mmd(a, b):
    return _mm(_b16(a), _b16(b))


def _mmdT(a, b):
    return _mmT(_b16(a), _b16(b))


def _mmx(a, b):
    return jax.lax.dot_general(
        a, b, (((1,), (0,)), ((), ())),
        preferred_element_type=jnp.float32,
        precision=jax.lax.Precision.HIGHEST,
    )


def _mmxT(a, b):
    return jax.lax.dot_general(
        a, b, (((1,), (1,)), ((), ())),
        preferred_element_type=jnp.float32,
        precision=jax.lax.Precision.HIGHEST,
    )


def _lnv(x, g, b):
    m = jnp.mean(x, axis=-1, keepdims=True)
    v = jnp.mean((x - m) ** 2, axis=-1, keepdims=True)
    return (x - m) / jnp.sqrt(v + 1e-5) * g + b


def _relu(x):
    return jnp.maximum(x, 0.0)


def _tc_body(*refs):
    (coords, amask, speeds, dist, ttg, C,
     te_W, te_b, te_g, te_be,
     gte_W, gte_b, gte_g, gte_be,
     res0_W, res0_b,
     Wq0, bq0, Wk0, bk0, Wv0, bv0, Ws0, bs0,
     Wq1, bq1, Wk1, bk1, Wv1, bv1, Ws1, bs1,
     Wq2, bq2, Wk2, bk2, Wv2, bv2, Ws2, bs2,
     Wq3, bq3, Wk3, bk3, Wv3, bv3, Ws3, bs3,
     gff_W, gff_b, gff_g, gff_be,
     gout_W, gout_b,
     aWq, abq, aWk, abk, aWv, abv, aWo, abo,
     ff_W, ff_b, ff_g, ff_be,
     as_W, as_b, as_g, as_be,
     ao_W, ao_b,
     cr_W1, cr_b1, cr_g, cr_be,
     cr_W2, cr_b2,
     logits, values,
     cur_a, cur_b, q_s, k_s, v_s, sf_s, res_s, ctx_s, act_s) = refs

    tev = _lnv(
        _relu(_r16(ttg[...]) * _r16(te_W[0:1, :]) + te_b[...]),
        te_g[...], te_be[...],
    )
    combined = jnp.concatenate(
        [coords[...], amask[...], speeds[...], dist[...], ttg[...], tev], axis=1
    )
    tfeat = _lnv(
        _relu(_mmd(tev[:, 61:64], gte_W[...]) + gte_b[...]),
        gte_g[...], gte_be[...],
    )

    def gnn_layer(cur_val, res_ref, nxt_ref, Wq, bq, Wk, bk, Wv, bv, Ws, bs):
        q_s[...] = _mmd(cur_val, Wq[...]) + bq[...]
        k_s[...] = _mmd(cur_val, Wk[...]) + bk[...]
        v_s[...] = _mmd(cur_val, Wv[...]) + bv[...]
        sf_s[...] = _mmd(cur_val, Ws[...]) + bs[...]

        def blk(b, _):
            sl = pl.ds(b * BLK, BLK)
            L = _mmxT(q_s[sl, :], k_s[...]) * 0.125
            Cb = C[sl, :]
            msk = Cb > 0.0
            m = jnp.max(jnp.where(msk, L, -1e30), axis=1, keepdims=True)
            m = jnp.where(m < -9e29, 0.0, m)
            e = Cb * jnp.exp(jnp.minimum(L - m, 0.0))
            ssum = jnp.sum(e, axis=1, keepdims=True) + 1e-16
            agg = _mmx(e, v_s[...]) * (1.0 / ssum)
            nxt_ref[sl, :] = _relu(agg + sf_s[sl, :]) + res_ref[sl, :]
            return 0

        jax.lax.fori_loop(0, NBLK, blk, 0)

    res_s[...] = _mmd(combined, res0_W[...]) + res0_b[...]
    gnn_layer(combined, res_s, cur_b, Wq0, bq0, Wk0, bk0, Wv0, bv0, Ws0, bs0)
    gnn_layer(cur_b[...], cur_b, cur_a, Wq1, bq1, Wk1, bk1, Wv1, bv1, Ws1, bs1)
    gnn_layer(cur_a[...], cur_a, cur_b, Wq2, bq2, Wk2, bk2, Wv2, bv2, Ws2, bs2)
    gnn_layer(cur_b[...], cur_b, cur_a, Wq3, bq3, Wk3, bk3, Wv3, bv3, Ws3, bs3)

    comb2 = jnp.concatenate([cur_a[...], tfeat], axis=1)
    fusedg = _lnv(
        _relu(_mmd(comb2, gff_W[...]) + gff_b[...]), gff_g[...], gff_be[...]
    )
    gnn_out = _mmd(fusedg, gout_W[...]) + gout_b[...]

    q_s[...] = _mmd(tev, aWq[...]) + abq[...]
    k_s[...] = _mmd(tev, aWk[...]) + abk[...]
    v_s[...] = _mmd(tev, aWv[...]) + abv[...]

    def ablk(b, _):
        sl = pl.ds(b * BLK, BLK)
        S = _mmdT(q_s[sl, :], k_s[...]) * 0.125
        m = jnp.max(S, axis=1, keepdims=True)
        e = jnp.exp(S - m)
        ctx_s[sl, :] = _mmd(e * (1.0 / jnp.sum(e, axis=1, keepdims=True)), v_s[...])
        return 0

    jax.lax.fori_loop(0, NBLK, ablk, 0)
    tctx = _mmd(ctx_s[...], aWo[...]) + abo[...]

    fused = _lnv(
        _relu(
            _mmd(jnp.concatenate([gnn_out, tev, tctx], axis=1), ff_W[...])
            + ff_b[...]
        ),
        ff_g[...], ff_be[...],
    )
    act_s[...] = _lnv(
        _relu(_mmd(fused, as_W[...]) + as_b[...]), as_g[...], as_be[...]
    )

    def lblk(b, _):
        sl = pl.ds(b * BLK, BLK)
        logits[sl, :] = _mmd(act_s[sl, :], ao_W[...]) + ao_b[...]
        return 0

    jax.lax.fori_loop(0, NBLK, lblk, 0)

    h = _lnv(
        _relu(_mmd(fused, cr_W1[...]) + cr_b1[...]), cr_g[...], cr_be[...]
    )
    values[...] = _mmd(h, cr_W2[...]) + cr_b2[...]


_TC_SCRATCH = [
    pltpu.VMEM((N, 64), jnp.float32),
    pltpu.VMEM((N, 64), jnp.float32),
    pltpu.VMEM((N, 64), jnp.float32),
    pltpu.VMEM((N, 64), jnp.float32),
    pltpu.VMEM((N, 64), jnp.float32),
    pltpu.VMEM((N, 64), jnp.float32),
    pltpu.VMEM((N, 64), jnp.float32),
    pltpu.VMEM((N, 64), jnp.float32),
    pltpu.VMEM((N, 128), jnp.float32),
]

_TC_OUT = [
    jax.ShapeDtypeStruct((N, N), jnp.float32),
    jax.ShapeDtypeStruct((N, 1), jnp.float32),
]


def _tc_args(coords, amask_f, speeds, dist, ttg, C, p):
    col = lambda v: v.reshape(N, 1)
    row = lambda v: v.reshape(1, -1)
    args = [coords, col(amask_f), col(speeds), col(dist), col(ttg), C,
            p["te_W"], row(p["te_b"]), row(p["te_g"]), row(p["te_beta"]),
            p["gte_W"], row(p["gte_b"]), row(p["gte_g"]), row(p["gte_beta"]),
            p["res0_W"], row(p["res0_b"])]
    for l in range(4):
        args += [p["conv_Wq"][l], row(p["conv_bq"][l]),
                 p["conv_Wk"][l], row(p["conv_bk"][l]),
                 p["conv_Wv"][l], row(p["conv_bv"][l]),
                 p["conv_Ws"][l], row(p["conv_bs"][l])]
    args += [p["gff_W"], row(p["gff_b"]), row(p["gff_g"]), row(p["gff_beta"]),
             p["gout_W"], row(p["gout_b"]),
             p["att_Wq"], row(p["att_bq"]), p["att_Wk"], row(p["att_bk"]),
             p["att_Wv"], row(p["att_bv"]), p["att_Wo"], row(p["att_bo"]),
             p["ff_W"], row(p["ff_b"]), row(p["ff_g"]), row(p["ff_beta"]),
             p["as_W"], row(p["as_b"]), row(p["as_g"]), row(p["as_beta"]),
             p["ao_W"], row(p["ao_b"]),
             p["cr_W1"], row(p["cr_b1"]), row(p["cr_g"]), row(p["cr_beta"]),
             p["cr_W2"], row(p["cr_b2"])]
    return args


_tc_forward = pl.pallas_call(
    _tc_body,
    out_shape=_TC_OUT,
    scratch_shapes=_TC_SCRATCH,
    compiler_params=pltpu.CompilerParams(
        vmem_limit_bytes=100 * 1024 * 1024
    ),
)


def kernel(mission_coords, edge_index, batch, uavs_info, action_mask,
           speeds, dist_matrix, timetogo_matrix, params):
    src = edge_index[0]
    dst = edge_index[1]
    C = _sc_counts()(src, dst)
    args = _tc_args(
        mission_coords, action_mask.astype(jnp.float32), speeds,
        dist_matrix, timetogo_matrix, C, params,
    )
    logits, values = _tc_forward(*args)
    return logits, values[:, 0]


# speedup vs baseline: 1.0407x; 1.0407x over previous
import functools

import jax
import jax.numpy as jnp
from jax.experimental import pallas as pl
from jax.experimental.pallas import tpu as pltpu
from jax.experimental.pallas import tpu_sc as plsc

N = 2048
E = 131072
NW = 32
ROWS_PER_W = N // NW
PASS_ROWS = ROWS_PER_W // 2
PASS_WORDS = PASS_ROWS * N
ECHUNK = 8192
BLK = 256
NBLK = N // BLK


NCHUNK = E // ECHUNK
UNROLL = 8


def _sc_counts_body(src_hbm, dst_hbm, c_hbm,
                    cblk, sbuf0, dbuf0, sbuf1, dbuf1, sem0, sem1):
    wid = jax.lax.axis_index("s") * 2 + jax.lax.axis_index("c")
    zeros16 = jnp.zeros((16,), jnp.float32)
    ones16 = jnp.ones((16,), jnp.float32)

    def start(c, sbuf, dbuf, sem):
        off = c * ECHUNK
        pltpu.make_async_copy(src_hbm.at[pl.ds(off, ECHUNK)], sbuf, sem).start()
        pltpu.make_async_copy(dst_hbm.at[pl.ds(off, ECHUNK)], dbuf, sem).start()

    def wait(sbuf, dbuf, sem):
        pltpu.make_async_copy(src_hbm.at[pl.ds(0, ECHUNK)], sbuf, sem).wait()
        pltpu.make_async_copy(dst_hbm.at[pl.ds(0, ECHUNK)], dbuf, sem).wait()

    for p in range(2):
        base = wid * ROWS_PER_W + p * PASS_ROWS
        start(0, sbuf0, dbuf0, sem0)

        def zbody(r, _):
            for u in range(N // (16 * UNROLL)):
                for v in range(UNROLL):
                    cblk[r, pl.ds((u * UNROLL + v) * 16, 16)] = zeros16
            return 0

        jax.lax.fori_loop(0, PASS_ROWS, zbody, 0)

        def process(sbuf, dbuf):
            def ibody(j, _):
                for u in range(UNROLL):
                    sl = pl.ds(j * (16 * UNROLL) + u * 16, 16)
                    d = dbuf[sl]
                    s = sbuf[sl]
                    rel = d - base
                    msk = rel.astype(jnp.uint32) < jnp.uint32(PASS_ROWS)
                    plsc.addupdate_scatter(cblk, [rel, s], ones16, mask=msk)
                return 0

            jax.lax.fori_loop(0, ECHUNK // (16 * UNROLL), ibody, 0)

        def cbody(i, _):
            c = i * 2
            start(c + 1, sbuf1, dbuf1, sem1)
            wait(sbuf0, dbuf0, sem0)
            process(sbuf0, dbuf0)

            @pl.when(c + 2 < NCHUNK)
            def _():
                start(c + 2, sbuf0, dbuf0, sem0)

            wait(sbuf1, dbuf1, sem1)
            process(sbuf1, dbuf1)
            return 0

        jax.lax.fori_loop(0, NCHUNK // 2, cbody, 0)
        pltpu.sync_copy(cblk, c_hbm.at[pl.ds(base, PASS_ROWS), :])


@functools.lru_cache(maxsize=1)
def _sc_counts():
    return pl.kernel(
        _sc_counts_body,
        out_type=jax.ShapeDtypeStruct((N, N), jnp.float32),
        mesh=plsc.VectorSubcoreMesh(
            core_axis_name="c", subcore_axis_name="s",
            num_cores=2, num_subcores=16,
        ),
        scratch_types=[
            pltpu.VMEM((PASS_ROWS, N), jnp.float32),
            pltpu.VMEM((ECHUNK,), jnp.int32),
            pltpu.VMEM((ECHUNK,), jnp.int32),
            pltpu.VMEM((ECHUNK,), jnp.int32),
            pltpu.VMEM((ECHUNK,), jnp.int32),
            pltpu.SemaphoreType.DMA,
            pltpu.SemaphoreType.DMA,
        ],
        compiler_params=pltpu.CompilerParams(needs_layout_passes=False),
    )


def _b16(x):
    return x.astype(jnp.bfloat16)


def _r16(x):
    return x.astype(jnp.bfloat16).astype(jnp.float32)


def _mm(a, b):
    return jax.lax.dot_general(
        a, b, (((1,), (0,)), ((), ())),
        preferred_element_type=jnp.float32,
    )


def _mmT(a, b):
    return jax.lax.dot_general(
        a, b, (((1,), (1,)), ((), ())),
        preferred_element_type=jnp.float32,
    )


def _mmd(a, b):
    return _mm(_b16(a), _b16(b))


def _mmdT(a, b):
    return _mmT(_b16(a), _b16(b))


def _mmx(a, b):
    return jax.lax.dot_general(
        a, b, (((1,), (0,)), ((), ())),
        preferred_element_type=jnp.float32,
        precision=jax.lax.Precision.HIGHEST,
    )


def _mmxT(a, b):
    return jax.lax.dot_general(
        a, b, (((1,), (1,)), ((), ())),
        preferred_element_type=jnp.float32,
        precision=jax.lax.Precision.HIGHEST,
    )


def _lnv(x, g, b):
    m = jnp.mean(x, axis=-1, keepdims=True)
    v = jnp.mean((x - m) ** 2, axis=-1, keepdims=True)
    return (x - m) / jnp.sqrt(v + 1e-5) * g + b


def _relu(x):
    return jnp.maximum(x, 0.0)


def _tc_body(*refs):
    (coords, amask, speeds, dist, ttg, C,
     te_W, te_b, te_g, te_be,
     gte_W, gte_b, gte_g, gte_be,
     res0_W, res0_b,
     Wq0, bq0, Wk0, bk0, Wv0, bv0, Ws0, bs0,
     Wq1, bq1, Wk1, bk1, Wv1, bv1, Ws1, bs1,
     Wq2, bq2, Wk2, bk2, Wv2, bv2, Ws2, bs2,
     Wq3, bq3, Wk3, bk3, Wv3, bv3, Ws3, bs3,
     gff_W, gff_b, gff_g, gff_be,
     gout_W, gout_b,
     aWq, abq, aWk, abk, aWv, abv, aWo, abo,
     ff_W, ff_b, ff_g, ff_be,
     as_W, as_b, as_g, as_be,
     ao_W, ao_b,
     cr_W1, cr_b1, cr_g, cr_be,
     cr_W2, cr_b2,
     logits, values,
     cur_a, cur_b, q_s, k_s, v_s, sf_s, res_s, ctx_s, act_s) = refs

    tev = _lnv(
        _relu(_r16(ttg[...]) * _r16(te_W[0:1, :]) + te_b[...]),
        te_g[...], te_be[...],
    )
    combined = jnp.concatenate(
        [coords[...], amask[...], speeds[...], dist[...], ttg[...], tev], axis=1
    )
    tfeat = _lnv(
        _relu(_mmd(tev[:, 61:64], gte_W[...]) + gte_b[...]),
        gte_g[...], gte_be[...],
    )

    def gnn_layer(cur_val, res_ref, nxt_ref, Wq, bq, Wk, bk, Wv, bv, Ws, bs):
        q_s[...] = _mmd(cur_val, Wq[...]) + bq[...]
        k_s[...] = _mmd(cur_val, Wk[...]) + bk[...]
        v_s[...] = _mmd(cur_val, Wv[...]) + bv[...]
        sf_s[...] = _mmd(cur_val, Ws[...]) + bs[...]

        def blk(b, _):
            sl = pl.ds(b * BLK, BLK)
            L = _mmxT(q_s[sl, :], k_s[...]) * 0.125
            Cb = C[sl, :]
            msk = Cb > 0.0
            m = jnp.max(jnp.where(msk, L, -1e30), axis=1, keepdims=True)
            m = jnp.where(m < -9e29, 0.0, m)
            e = Cb * jnp.exp(jnp.minimum(L - m, 0.0))
            ssum = jnp.sum(e, axis=1, keepdims=True) + 1e-16
            agg = _mmx(e, v_s[...]) * (1.0 / ssum)
            nxt_ref[sl, :] = _relu(agg + sf_s[sl, :]) + res_ref[sl, :]
            return 0

        jax.lax.fori_loop(0, NBLK, blk, 0)

    res_s[...] = _mmd(combined, res0_W[...]) + res0_b[...]
    gnn_layer(combined, res_s, cur_b, Wq0, bq0, Wk0, bk0, Wv0, bv0, Ws0, bs0)
    gnn_layer(cur_b[...], cur_b, cur_a, Wq1, bq1, Wk1, bk1, Wv1, bv1, Ws1, bs1)
    gnn_layer(cur_a[...], cur_a, cur_b, Wq2, bq2, Wk2, bk2, Wv2, bv2, Ws2, bs2)
    gnn_layer(cur_b[...], cur_b, cur_a, Wq3, bq3, Wk3, bk3, Wv3, bv3, Ws3, bs3)

    comb2 = jnp.concatenate([cur_a[...], tfeat], axis=1)
    fusedg = _lnv(
        _relu(_mmd(comb2, gff_W[...]) + gff_b[...]), gff_g[...], gff_be[...]
    )
    gnn_out = _mmd(fusedg, gout_W[...]) + gout_b[...]

    q_s[...] = _mmd(tev, aWq[...]) + abq[...]
    k_s[...] = _mmd(tev, aWk[...]) + abk[...]
    v_s[...] = _mmd(tev, aWv[...]) + abv[...]

    def ablk(b, _):
        sl = pl.ds(b * BLK, BLK)
        S = _mmdT(q_s[sl, :], k_s[...]) * 0.125
        m = jnp.max(S, axis=1, keepdims=True)
        e = jnp.exp(S - m)
        ctx_s[sl, :] = _mmd(e * (1.0 / jnp.sum(e, axis=1, keepdims=True)), v_s[...])
        return 0

    jax.lax.fori_loop(0, NBLK, ablk, 0)
    tctx = _mmd(ctx_s[...], aWo[...]) + abo[...]

    fused = _lnv(
        _relu(
            _mmd(jnp.concatenate([gnn_out, tev, tctx], axis=1), ff_W[...])
            + ff_b[...]
        ),
        ff_g[...], ff_be[...],
    )
    act_s[...] = _lnv(
        _relu(_mmd(fused, as_W[...]) + as_b[...]), as_g[...], as_be[...]
    )

    def lblk(b, _):
        sl = pl.ds(b * BLK, BLK)
        logits[sl, :] = _mmd(act_s[sl, :], ao_W[...]) + ao_b[...]
        return 0

    jax.lax.fori_loop(0, NBLK, lblk, 0)

    h = _lnv(
        _relu(_mmd(fused, cr_W1[...]) + cr_b1[...]), cr_g[...], cr_be[...]
    )
    values[...] = _mmd(h, cr_W2[...]) + cr_b2[...]


_TC_SCRATCH = [
    pltpu.VMEM((N, 64), jnp.float32),
    pltpu.VMEM((N, 64), jnp.float32),
    pltpu.VMEM((N, 64), jnp.float32),
    pltpu.VMEM((N, 64), jnp.float32),
    pltpu.VMEM((N, 64), jnp.float32),
    pltpu.VMEM((N, 64), jnp.float32),
    pltpu.VMEM((N, 64), jnp.float32),
    pltpu.VMEM((N, 64), jnp.float32),
    pltpu.VMEM((N, 128), jnp.float32),
]

_TC_OUT = [
    jax.ShapeDtypeStruct((N, N), jnp.float32),
    jax.ShapeDtypeStruct((N, 1), jnp.float32),
]


def _tc_args(coords, amask_f, speeds, dist, ttg, C, p):
    col = lambda v: v.reshape(N, 1)
    row = lambda v: v.reshape(1, -1)
    args = [coords, col(amask_f), col(speeds), col(dist), col(ttg), C,
            p["te_W"], row(p["te_b"]), row(p["te_g"]), row(p["te_beta"]),
            p["gte_W"], row(p["gte_b"]), row(p["gte_g"]), row(p["gte_beta"]),
            p["res0_W"], row(p["res0_b"])]
    for l in range(4):
        args += [p["conv_Wq"][l], row(p["conv_bq"][l]),
                 p["conv_Wk"][l], row(p["conv_bk"][l]),
                 p["conv_Wv"][l], row(p["conv_bv"][l]),
                 p["conv_Ws"][l], row(p["conv_bs"][l])]
    args += [p["gff_W"], row(p["gff_b"]), row(p["gff_g"]), row(p["gff_beta"]),
             p["gout_W"], row(p["gout_b"]),
             p["att_Wq"], row(p["att_bq"]), p["att_Wk"], row(p["att_bk"]),
             p["att_Wv"], row(p["att_bv"]), p["att_Wo"], row(p["att_bo"]),
             p["ff_W"], row(p["ff_b"]), row(p["ff_g"]), row(p["ff_beta"]),
             p["as_W"], row(p["as_b"]), row(p["as_g"]), row(p["as_beta"]),
             p["ao_W"], row(p["ao_b"]),
             p["cr_W1"], row(p["cr_b1"]), row(p["cr_g"]), row(p["cr_beta"]),
             p["cr_W2"], row(p["cr_b2"])]
    return args


_tc_forward = pl.pallas_call(
    _tc_body,
    out_shape=_TC_OUT,
    scratch_shapes=_TC_SCRATCH,
    compiler_params=pltpu.CompilerParams(
        vmem_limit_bytes=100 * 1024 * 1024
    ),
)


def kernel(mission_coords, edge_index, batch, uavs_info, action_mask,
           speeds, dist_matrix, timetogo_matrix, params):
    src = edge_index[0]
    dst = edge_index[1]
    C = _sc_counts()(src, dst)
    args = _tc_args(
        mission_coords, action_mask.astype(jnp.float32), speeds,
        dist_matrix, timetogo_matrix, C, params,
    )
    logits, values = _tc_forward(*args)
    return logits, values[:, 0]


# speedup vs baseline: 1.5600x; 1.4990x over previous
import functools

import jax
import jax.numpy as jnp
from jax.experimental import pallas as pl
from jax.experimental.pallas import tpu as pltpu
from jax.experimental.pallas import tpu_sc as plsc

N = 2048
E = 131072
NW = 32
ROWS_PER_W = N // NW
PASS_ROWS = ROWS_PER_W // 2
PASS_WORDS = PASS_ROWS * N
ECHUNK = 8192
BLK = 256
NBLK = N // BLK


NCHUNK = E // ECHUNK
UNROLL = 8


def _sc_counts_body(src_hbm, dst_hbm, c_hbm,
                    cblk, sbuf0, dbuf0, sbuf1, dbuf1, sem0, sem1):
    wid = jax.lax.axis_index("s") * 2 + jax.lax.axis_index("c")
    zeros16 = jnp.zeros((16,), jnp.float32)
    ones16 = jnp.ones((16,), jnp.float32)

    def start(c, sbuf, dbuf, sem):
        off = c * ECHUNK
        pltpu.make_async_copy(src_hbm.at[pl.ds(off, ECHUNK)], sbuf, sem).start()
        pltpu.make_async_copy(dst_hbm.at[pl.ds(off, ECHUNK)], dbuf, sem).start()

    def wait(sbuf, dbuf, sem):
        pltpu.make_async_copy(src_hbm.at[pl.ds(0, ECHUNK)], sbuf, sem).wait()
        pltpu.make_async_copy(dst_hbm.at[pl.ds(0, ECHUNK)], dbuf, sem).wait()

    for p in range(2):
        base = wid * ROWS_PER_W + p * PASS_ROWS
        start(0, sbuf0, dbuf0, sem0)

        def zbody(r, _):
            for u in range(N // (16 * UNROLL)):
                for v in range(UNROLL):
                    cblk[r, pl.ds((u * UNROLL + v) * 16, 16)] = zeros16
            return 0

        jax.lax.fori_loop(0, PASS_ROWS, zbody, 0)

        def process(sbuf, dbuf):
            def ibody(j, _):
                for u in range(UNROLL):
                    sl = pl.ds(j * (16 * UNROLL) + u * 16, 16)
                    d = dbuf[sl]
                    s = sbuf[sl]
                    rel = d - base
                    msk = rel.astype(jnp.uint32) < jnp.uint32(PASS_ROWS)
                    plsc.addupdate_scatter(cblk, [rel, s], ones16, mask=msk)
                return 0

            jax.lax.fori_loop(0, ECHUNK // (16 * UNROLL), ibody, 0)

        def cbody(i, _):
            c = i * 2
            start(c + 1, sbuf1, dbuf1, sem1)
            wait(sbuf0, dbuf0, sem0)
            process(sbuf0, dbuf0)

            @pl.when(c + 2 < NCHUNK)
            def _():
                start(c + 2, sbuf0, dbuf0, sem0)

            wait(sbuf1, dbuf1, sem1)
            process(sbuf1, dbuf1)
            return 0

        jax.lax.fori_loop(0, NCHUNK // 2, cbody, 0)
        pltpu.sync_copy(cblk, c_hbm.at[pl.ds(base, PASS_ROWS), :])


@functools.lru_cache(maxsize=1)
def _sc_counts():
    return pl.kernel(
        _sc_counts_body,
        out_type=jax.ShapeDtypeStruct((N, N), jnp.float32),
        mesh=plsc.VectorSubcoreMesh(
            core_axis_name="c", subcore_axis_name="s",
            num_cores=2, num_subcores=16,
        ),
        scratch_types=[
            pltpu.VMEM((PASS_ROWS, N), jnp.float32),
            pltpu.VMEM((ECHUNK,), jnp.int32),
            pltpu.VMEM((ECHUNK,), jnp.int32),
            pltpu.VMEM((ECHUNK,), jnp.int32),
            pltpu.VMEM((ECHUNK,), jnp.int32),
            pltpu.SemaphoreType.DMA,
            pltpu.SemaphoreType.DMA,
        ],
        compiler_params=pltpu.CompilerParams(needs_layout_passes=False),
    )


def _b16(x):
    return x.astype(jnp.bfloat16)


def _r16(x):
    return x.astype(jnp.bfloat16).astype(jnp.float32)


def _mm(a, b):
    return jax.lax.dot_general(
        a, b, (((1,), (0,)), ((), ())),
        preferred_element_type=jnp.float32,
    )


def _mmT(a, b):
    return jax.lax.dot_general(
        a, b, (((1,), (1,)), ((), ())),
        preferred_element_type=jnp.float32,
    )


def _mmd(a, b):
    return _mm(_b16(a), _b16(b))


def _mmdT(a, b):
    return _mmT(_b16(a), _b16(b))


def _split16(x):
    hi = _b16(x)
    lo = _b16(x - hi.astype(jnp.float32))
    return hi, lo


def _mmx(a, b):
    ah, al = _split16(a)
    bh, bl = _split16(b)
    return _mm(ah, bh) + (_mm(ah, bl) + _mm(al, bh))


def _mmxT(a, b):
    ah, al = _split16(a)
    bh, bl = _split16(b)
    return _mmT(ah, bh) + (_mmT(ah, bl) + _mmT(al, bh))


def _lnv(x, g, b):
    m = jnp.mean(x, axis=-1, keepdims=True)
    v = jnp.mean((x - m) ** 2, axis=-1, keepdims=True)
    return (x - m) / jnp.sqrt(v + 1e-5) * g + b


def _relu(x):
    return jnp.maximum(x, 0.0)


def _tc_body(*refs):
    (coords, amask, speeds, dist, ttg, C,
     te_W, te_b, te_g, te_be,
     gte_W, gte_b, gte_g, gte_be,
     res0_W, res0_b,
     Wq0, bq0, Wk0, bk0, Wv0, bv0, Ws0, bs0,
     Wq1, bq1, Wk1, bk1, Wv1, bv1, Ws1, bs1,
     Wq2, bq2, Wk2, bk2, Wv2, bv2, Ws2, bs2,
     Wq3, bq3, Wk3, bk3, Wv3, bv3, Ws3, bs3,
     gff_W, gff_b, gff_g, gff_be,
     gout_W, gout_b,
     aWq, abq, aWk, abk, aWv, abv, aWo, abo,
     ff_W, ff_b, ff_g, ff_be,
     as_W, as_b, as_g, as_be,
     ao_W, ao_b,
     cr_W1, cr_b1, cr_g, cr_be,
     cr_W2, cr_b2,
     logits, values,
     cur_a, cur_b, q_s, k_s, v_s, sf_s, res_s, ctx_s, act_s) = refs

    tev = _lnv(
        _relu(_r16(ttg[...]) * _r16(te_W[0:1, :]) + te_b[...]),
        te_g[...], te_be[...],
    )
    combined = jnp.concatenate(
        [coords[...], amask[...], speeds[...], dist[...], ttg[...], tev], axis=1
    )
    tfeat = _lnv(
        _relu(_mmd(tev[:, 61:64], gte_W[...]) + gte_b[...]),
        gte_g[...], gte_be[...],
    )

    def gnn_layer(cur_val, res_ref, nxt_ref, Wq, bq, Wk, bk, Wv, bv, Ws, bs):
        q_s[...] = _mmd(cur_val, Wq[...]) + bq[...]
        k_s[...] = _mmd(cur_val, Wk[...]) + bk[...]
        v_s[...] = _mmd(cur_val, Wv[...]) + bv[...]
        sf_s[...] = _mmd(cur_val, Ws[...]) + bs[...]

        def blk(b, _):
            sl = pl.ds(b * BLK, BLK)
            L = _mmxT(q_s[sl, :], k_s[...]) * 0.125
            Cb = C[sl, :]
            msk = Cb > 0.0
            m = jnp.max(jnp.where(msk, L, -1e30), axis=1, keepdims=True)
            m = jnp.where(m < -9e29, 0.0, m)
            e = Cb * jnp.exp(jnp.minimum(L - m, 0.0))
            ssum = jnp.sum(e, axis=1, keepdims=True) + 1e-16
            agg = _mmx(e, v_s[...]) * (1.0 / ssum)
            nxt_ref[sl, :] = _relu(agg + sf_s[sl, :]) + res_ref[sl, :]
            return 0

        jax.lax.fori_loop(0, NBLK, blk, 0)

    res_s[...] = _mmd(combined, res0_W[...]) + res0_b[...]
    gnn_layer(combined, res_s, cur_b, Wq0, bq0, Wk0, bk0, Wv0, bv0, Ws0, bs0)
    gnn_layer(cur_b[...], cur_b, cur_a, Wq1, bq1, Wk1, bk1, Wv1, bv1, Ws1, bs1)
    gnn_layer(cur_a[...], cur_a, cur_b, Wq2, bq2, Wk2, bk2, Wv2, bv2, Ws2, bs2)
    gnn_layer(cur_b[...], cur_b, cur_a, Wq3, bq3, Wk3, bk3, Wv3, bv3, Ws3, bs3)

    comb2 = jnp.concatenate([cur_a[...], tfeat], axis=1)
    fusedg = _lnv(
        _relu(_mmd(comb2, gff_W[...]) + gff_b[...]), gff_g[...], gff_be[...]
    )
    gnn_out = _mmd(fusedg, gout_W[...]) + gout_b[...]

    q_s[...] = _mmd(tev, aWq[...]) + abq[...]
    k_s[...] = _mmd(tev, aWk[...]) + abk[...]
    v_s[...] = _mmd(tev, aWv[...]) + abv[...]

    def ablk(b, _):
        sl = pl.ds(b * BLK, BLK)
        S = _mmdT(q_s[sl, :], k_s[...]) * 0.125
        m = jnp.max(S, axis=1, keepdims=True)
        e = jnp.exp(S - m)
        ctx_s[sl, :] = _mmd(e * (1.0 / jnp.sum(e, axis=1, keepdims=True)), v_s[...])
        return 0

    jax.lax.fori_loop(0, NBLK, ablk, 0)
    tctx = _mmd(ctx_s[...], aWo[...]) + abo[...]

    fused = _lnv(
        _relu(
            _mmd(jnp.concatenate([gnn_out, tev, tctx], axis=1), ff_W[...])
            + ff_b[...]
        ),
        ff_g[...], ff_be[...],
    )
    act_s[...] = _lnv(
        _relu(_mmd(fused, as_W[...]) + as_b[...]), as_g[...], as_be[...]
    )

    def lblk(b, _):
        sl = pl.ds(b * BLK, BLK)
        logits[sl, :] = _mmd(act_s[sl, :], ao_W[...]) + ao_b[...]
        return 0

    jax.lax.fori_loop(0, NBLK, lblk, 0)

    h = _lnv(
        _relu(_mmd(fused, cr_W1[...]) + cr_b1[...]), cr_g[...], cr_be[...]
    )
    values[...] = _mmd(h, cr_W2[...]) + cr_b2[...]


_TC_SCRATCH = [
    pltpu.VMEM((N, 64), jnp.float32),
    pltpu.VMEM((N, 64), jnp.float32),
    pltpu.VMEM((N, 64), jnp.float32),
    pltpu.VMEM((N, 64), jnp.float32),
    pltpu.VMEM((N, 64), jnp.float32),
    pltpu.VMEM((N, 64), jnp.float32),
    pltpu.VMEM((N, 64), jnp.float32),
    pltpu.VMEM((N, 64), jnp.float32),
    pltpu.VMEM((N, 128), jnp.float32),
]

_TC_OUT = [
    jax.ShapeDtypeStruct((N, N), jnp.float32),
    jax.ShapeDtypeStruct((N, 1), jnp.float32),
]


def _tc_args(coords, amask_f, speeds, dist, ttg, C, p):
    col = lambda v: v.reshape(N, 1)
    row = lambda v: v.reshape(1, -1)
    args = [coords, col(amask_f), col(speeds), col(dist), col(ttg), C,
            p["te_W"], row(p["te_b"]), row(p["te_g"]), row(p["te_beta"]),
            p["gte_W"], row(p["gte_b"]), row(p["gte_g"]), row(p["gte_beta"]),
            p["res0_W"], row(p["res0_b"])]
    for l in range(4):
        args += [p["conv_Wq"][l], row(p["conv_bq"][l]),
                 p["conv_Wk"][l], row(p["conv_bk"][l]),
                 p["conv_Wv"][l], row(p["conv_bv"][l]),
                 p["conv_Ws"][l], row(p["conv_bs"][l])]
    args += [p["gff_W"], row(p["gff_b"]), row(p["gff_g"]), row(p["gff_beta"]),
             p["gout_W"], row(p["gout_b"]),
             p["att_Wq"], row(p["att_bq"]), p["att_Wk"], row(p["att_bk"]),
             p["att_Wv"], row(p["att_bv"]), p["att_Wo"], row(p["att_bo"]),
             p["ff_W"], row(p["ff_b"]), row(p["ff_g"]), row(p["ff_beta"]),
             p["as_W"], row(p["as_b"]), row(p["as_g"]), row(p["as_beta"]),
             p["ao_W"], row(p["ao_b"]),
             p["cr_W1"], row(p["cr_b1"]), row(p["cr_g"]), row(p["cr_beta"]),
             p["cr_W2"], row(p["cr_b2"])]
    return args


_tc_forward = pl.pallas_call(
    _tc_body,
    out_shape=_TC_OUT,
    scratch_shapes=_TC_SCRATCH,
    compiler_params=pltpu.CompilerParams(
        vmem_limit_bytes=100 * 1024 * 1024
    ),
)


def kernel(mission_coords, edge_index, batch, uavs_info, action_mask,
           speeds, dist_matrix, timetogo_matrix, params):
    src = edge_index[0]
    dst = edge_index[1]
    C = _sc_counts()(src, dst)
    args = _tc_args(
        mission_coords, action_mask.astype(jnp.float32), speeds,
        dist_matrix, timetogo_matrix, C, params,
    )
    logits, values = _tc_forward(*args)
    return logits, values[:, 0]


# speedup vs baseline: 1.6131x; 1.0341x over previous
import functools

import jax
import jax.numpy as jnp
from jax.experimental import pallas as pl
from jax.experimental.pallas import tpu as pltpu
from jax.experimental.pallas import tpu_sc as plsc

N = 2048
E = 131072
NW = 32
ROWS_PER_W = N // NW
PASS_ROWS = ROWS_PER_W // 2
PASS_WORDS = PASS_ROWS * N
ECHUNK = 8192
BLK = 256
NBLK = N // BLK


NCHUNK = E // ECHUNK
UNROLL = 8


def _sc_counts_body(src_hbm, dst_hbm, c_hbm,
                    cblk, sbuf0, dbuf0, sbuf1, dbuf1, sem0, sem1):
    wid = jax.lax.axis_index("s") * 2 + jax.lax.axis_index("c")
    zeros16 = jnp.zeros((16,), jnp.float32)
    ones16 = jnp.ones((16,), jnp.float32)

    def start(c, sbuf, dbuf, sem):
        off = c * ECHUNK
        pltpu.make_async_copy(src_hbm.at[pl.ds(off, ECHUNK)], sbuf, sem).start()
        pltpu.make_async_copy(dst_hbm.at[pl.ds(off, ECHUNK)], dbuf, sem).start()

    def wait(sbuf, dbuf, sem):
        pltpu.make_async_copy(src_hbm.at[pl.ds(0, ECHUNK)], sbuf, sem).wait()
        pltpu.make_async_copy(dst_hbm.at[pl.ds(0, ECHUNK)], dbuf, sem).wait()

    for p in range(2):
        base = wid * ROWS_PER_W + p * PASS_ROWS
        start(0, sbuf0, dbuf0, sem0)

        def zbody(r, _):
            for u in range(N // (16 * UNROLL)):
                for v in range(UNROLL):
                    cblk[pl.ds(r * N + (u * UNROLL + v) * 16, 16)] = zeros16
            return 0

        jax.lax.fori_loop(0, PASS_ROWS, zbody, 0)

        def process(sbuf, dbuf):
            def ibody(j, _):
                for u in range(UNROLL):
                    sl = pl.ds(j * (16 * UNROLL) + u * 16, 16)
                    d = dbuf[sl]
                    s = sbuf[sl]
                    rel = d - base
                    msk = rel.astype(jnp.uint32) < jnp.uint32(PASS_ROWS)
                    idx = (rel << 11) + s
                    plsc.addupdate_scatter(cblk, [idx], ones16, mask=msk)
                return 0

            jax.lax.fori_loop(0, ECHUNK // (16 * UNROLL), ibody, 0)

        def cbody(i, _):
            c = i * 2
            start(c + 1, sbuf1, dbuf1, sem1)
            wait(sbuf0, dbuf0, sem0)
            process(sbuf0, dbuf0)

            @pl.when(c + 2 < NCHUNK)
            def _():
                start(c + 2, sbuf0, dbuf0, sem0)

            wait(sbuf1, dbuf1, sem1)
            process(sbuf1, dbuf1)
            return 0

        jax.lax.fori_loop(0, NCHUNK // 2, cbody, 0)
        pltpu.sync_copy(cblk, c_hbm.at[pl.ds(base * N, PASS_WORDS)])


@functools.lru_cache(maxsize=1)
def _sc_counts():
    return pl.kernel(
        _sc_counts_body,
        out_type=jax.ShapeDtypeStruct((N * N,), jnp.float32),
        mesh=plsc.VectorSubcoreMesh(
            core_axis_name="c", subcore_axis_name="s",
            num_cores=2, num_subcores=16,
        ),
        scratch_types=[
            pltpu.VMEM((PASS_WORDS,), jnp.float32),
            pltpu.VMEM((ECHUNK,), jnp.int32),
            pltpu.VMEM((ECHUNK,), jnp.int32),
            pltpu.VMEM((ECHUNK,), jnp.int32),
            pltpu.VMEM((ECHUNK,), jnp.int32),
            pltpu.SemaphoreType.DMA,
            pltpu.SemaphoreType.DMA,
        ],
        compiler_params=pltpu.CompilerParams(needs_layout_passes=False),
    )


def _b16(x):
    return x.astype(jnp.bfloat16)


def _r16(x):
    return x.astype(jnp.bfloat16).astype(jnp.float32)


def _mm(a, b):
    return jax.lax.dot_general(
        a, b, (((1,), (0,)), ((), ())),
        preferred_element_type=jnp.float32,
    )


def _mmT(a, b):
    return jax.lax.dot_general(
        a, b, (((1,), (1,)), ((), ())),
        preferred_element_type=jnp.float32,
    )


def _mmd(a, b):
    return _mm(_b16(a), _b16(b))


def _mmdT(a, b):
    return _mmT(_b16(a), _b16(b))


def _split16(x):
    hi = _b16(x)
    lo = _b16(x - hi.astype(jnp.float32))
    return hi, lo


def _mmx(a, b):
    ah, al = _split16(a)
    bh, bl = _split16(b)
    return _mm(ah, bh) + (_mm(ah, bl) + _mm(al, bh))


def _mmxT(a, b):
    ah, al = _split16(a)
    bh, bl = _split16(b)
    return _mmT(ah, bh) + (_mmT(ah, bl) + _mmT(al, bh))


def _lnv(x, g, b):
    m = jnp.mean(x, axis=-1, keepdims=True)
    v = jnp.mean((x - m) ** 2, axis=-1, keepdims=True)
    return (x - m) / jnp.sqrt(v + 1e-5) * g + b


def _relu(x):
    return jnp.maximum(x, 0.0)


def _tc_body(*refs):
    (coords, amask, speeds, dist, ttg, C,
     te_W, te_b, te_g, te_be,
     gte_W, gte_b, gte_g, gte_be,
     res0_W, res0_b,
     Wq0, bq0, Wk0, bk0, Wv0, bv0, Ws0, bs0,
     Wq1, bq1, Wk1, bk1, Wv1, bv1, Ws1, bs1,
     Wq2, bq2, Wk2, bk2, Wv2, bv2, Ws2, bs2,
     Wq3, bq3, Wk3, bk3, Wv3, bv3, Ws3, bs3,
     gff_W, gff_b, gff_g, gff_be,
     gout_W, gout_b,
     aWq, abq, aWk, abk, aWv, abv, aWo, abo,
     ff_W, ff_b, ff_g, ff_be,
     as_W, as_b, as_g, as_be,
     ao_W, ao_b,
     cr_W1, cr_b1, cr_g, cr_be,
     cr_W2, cr_b2,
     logits, values,
     cur_a, cur_b, q_s, k_s, v_s, sf_s, res_s, ctx_s, act_s) = refs

    tev = _lnv(
        _relu(_r16(ttg[...]) * _r16(te_W[0:1, :]) + te_b[...]),
        te_g[...], te_be[...],
    )
    combined = jnp.concatenate(
        [coords[...], amask[...], speeds[...], dist[...], ttg[...], tev], axis=1
    )
    tfeat = _lnv(
        _relu(_mmd(tev[:, 61:64], gte_W[...]) + gte_b[...]),
        gte_g[...], gte_be[...],
    )

    def gnn_layer(cur_val, res_ref, nxt_ref, Wq, bq, Wk, bk, Wv, bv, Ws, bs):
        q_s[...] = _mmd(cur_val, Wq[...]) + bq[...]
        k_s[...] = _mmd(cur_val, Wk[...]) + bk[...]
        v_s[...] = _mmd(cur_val, Wv[...]) + bv[...]
        sf_s[...] = _mmd(cur_val, Ws[...]) + bs[...]

        def blk(b, _):
            sl = pl.ds(b * BLK, BLK)
            L = _mmxT(q_s[sl, :], k_s[...]) * 0.125
            Cb = C[sl, :]
            msk = Cb > 0.0
            m = jnp.max(jnp.where(msk, L, -1e30), axis=1, keepdims=True)
            m = jnp.where(m < -9e29, 0.0, m)
            e = Cb * jnp.exp(jnp.minimum(L - m, 0.0))
            ssum = jnp.sum(e, axis=1, keepdims=True) + 1e-16
            agg = _mmx(e, v_s[...]) * (1.0 / ssum)
            nxt_ref[sl, :] = _relu(agg + sf_s[sl, :]) + res_ref[sl, :]
            return 0

        jax.lax.fori_loop(0, NBLK, blk, 0)

    res_s[...] = _mmd(combined, res0_W[...]) + res0_b[...]
    gnn_layer(combined, res_s, cur_b, Wq0, bq0, Wk0, bk0, Wv0, bv0, Ws0, bs0)
    gnn_layer(cur_b[...], cur_b, cur_a, Wq1, bq1, Wk1, bk1, Wv1, bv1, Ws1, bs1)
    gnn_layer(cur_a[...], cur_a, cur_b, Wq2, bq2, Wk2, bk2, Wv2, bv2, Ws2, bs2)
    gnn_layer(cur_b[...], cur_b, cur_a, Wq3, bq3, Wk3, bk3, Wv3, bv3, Ws3, bs3)

    comb2 = jnp.concatenate([cur_a[...], tfeat], axis=1)
    fusedg = _lnv(
        _relu(_mmd(comb2, gff_W[...]) + gff_b[...]), gff_g[...], gff_be[...]
    )
    gnn_out = _mmd(fusedg, gout_W[...]) + gout_b[...]

    q_s[...] = _mmd(tev, aWq[...]) + abq[...]
    k_s[...] = _mmd(tev, aWk[...]) + abk[...]
    v_s[...] = _mmd(tev, aWv[...]) + abv[...]

    def ablk(b, _):
        sl = pl.ds(b * BLK, BLK)
        S = _mmdT(q_s[sl, :], k_s[...]) * 0.125
        m = jnp.max(S, axis=1, keepdims=True)
        e = jnp.exp(S - m)
        ctx_s[sl, :] = _mmd(e * (1.0 / jnp.sum(e, axis=1, keepdims=True)), v_s[...])
        return 0

    jax.lax.fori_loop(0, NBLK, ablk, 0)
    tctx = _mmd(ctx_s[...], aWo[...]) + abo[...]

    fused = _lnv(
        _relu(
            _mmd(jnp.concatenate([gnn_out, tev, tctx], axis=1), ff_W[...])
            + ff_b[...]
        ),
        ff_g[...], ff_be[...],
    )
    act_s[...] = _lnv(
        _relu(_mmd(fused, as_W[...]) + as_b[...]), as_g[...], as_be[...]
    )

    def lblk(b, _):
        sl = pl.ds(b * BLK, BLK)
        logits[sl, :] = _mmd(act_s[sl, :], ao_W[...]) + ao_b[...]
        return 0

    jax.lax.fori_loop(0, NBLK, lblk, 0)

    h = _lnv(
        _relu(_mmd(fused, cr_W1[...]) + cr_b1[...]), cr_g[...], cr_be[...]
    )
    values[...] = _mmd(h, cr_W2[...]) + cr_b2[...]


_TC_SCRATCH = [
    pltpu.VMEM((N, 64), jnp.float32),
    pltpu.VMEM((N, 64), jnp.float32),
    pltpu.VMEM((N, 64), jnp.float32),
    pltpu.VMEM((N, 64), jnp.float32),
    pltpu.VMEM((N, 64), jnp.float32),
    pltpu.VMEM((N, 64), jnp.float32),
    pltpu.VMEM((N, 64), jnp.float32),
    pltpu.VMEM((N, 64), jnp.float32),
    pltpu.VMEM((N, 128), jnp.float32),
]

_TC_OUT = [
    jax.ShapeDtypeStruct((N, N), jnp.float32),
    jax.ShapeDtypeStruct((N, 1), jnp.float32),
]


def _tc_args(coords, amask_f, speeds, dist, ttg, C, p):
    col = lambda v: v.reshape(N, 1)
    row = lambda v: v.reshape(1, -1)
    args = [coords, col(amask_f), col(speeds), col(dist), col(ttg), C,
            p["te_W"], row(p["te_b"]), row(p["te_g"]), row(p["te_beta"]),
            p["gte_W"], row(p["gte_b"]), row(p["gte_g"]), row(p["gte_beta"]),
            p["res0_W"], row(p["res0_b"])]
    for l in range(4):
        args += [p["conv_Wq"][l], row(p["conv_bq"][l]),
                 p["conv_Wk"][l], row(p["conv_bk"][l]),
                 p["conv_Wv"][l], row(p["conv_bv"][l]),
                 p["conv_Ws"][l], row(p["conv_bs"][l])]
    args += [p["gff_W"], row(p["gff_b"]), row(p["gff_g"]), row(p["gff_beta"]),
             p["gout_W"], row(p["gout_b"]),
             p["att_Wq"], row(p["att_bq"]), p["att_Wk"], row(p["att_bk"]),
             p["att_Wv"], row(p["att_bv"]), p["att_Wo"], row(p["att_bo"]),
             p["ff_W"], row(p["ff_b"]), row(p["ff_g"]), row(p["ff_beta"]),
             p["as_W"], row(p["as_b"]), row(p["as_g"]), row(p["as_beta"]),
             p["ao_W"], row(p["ao_b"]),
             p["cr_W1"], row(p["cr_b1"]), row(p["cr_g"]), row(p["cr_beta"]),
             p["cr_W2"], row(p["cr_b2"])]
    return args


_tc_forward = pl.pallas_call(
    _tc_body,
    out_shape=_TC_OUT,
    scratch_shapes=_TC_SCRATCH,
    compiler_params=pltpu.CompilerParams(
        vmem_limit_bytes=100 * 1024 * 1024
    ),
)


def kernel(mission_coords, edge_index, batch, uavs_info, action_mask,
           speeds, dist_matrix, timetogo_matrix, params):
    src = edge_index[0]
    dst = edge_index[1]
    C = _sc_counts()(src, dst).reshape(N, N)
    args = _tc_args(
        mission_coords, action_mask.astype(jnp.float32), speeds,
        dist_matrix, timetogo_matrix, C, params,
    )
    logits, values = _tc_forward(*args)
    return logits, values[:, 0]


# speedup vs baseline: 2.0971x; 1.3000x over previous
import functools

import jax
import jax.numpy as jnp
from jax.experimental import pallas as pl
from jax.experimental.pallas import tpu as pltpu
from jax.experimental.pallas import tpu_sc as plsc

N = 2048
E = 131072
NW = 32
ROWS_PER_W = N // NW
PASS_ROWS = ROWS_PER_W // 2
PASS_WORDS = PASS_ROWS * N
ECHUNK = 8192
BLK = 256
NBLK = N // BLK


NCHUNK = E // ECHUNK
UNROLL = 8


def _sc_counts_body(src_hbm, dst_hbm, c_hbm,
                    cblk, sbuf0, dbuf0, sbuf1, dbuf1, sem0, sem1):
    wid = jax.lax.axis_index("s") * 2 + jax.lax.axis_index("c")
    zeros16 = jnp.zeros((16,), jnp.float32)
    ones16 = jnp.ones((16,), jnp.float32)

    def start(c, sbuf, dbuf, sem):
        off = c * ECHUNK
        pltpu.make_async_copy(src_hbm.at[pl.ds(off, ECHUNK)], sbuf, sem).start()
        pltpu.make_async_copy(dst_hbm.at[pl.ds(off, ECHUNK)], dbuf, sem).start()

    def wait(sbuf, dbuf, sem):
        pltpu.make_async_copy(src_hbm.at[pl.ds(0, ECHUNK)], sbuf, sem).wait()
        pltpu.make_async_copy(dst_hbm.at[pl.ds(0, ECHUNK)], dbuf, sem).wait()

    for p in range(2):
        base = wid * ROWS_PER_W + p * PASS_ROWS
        start(0, sbuf0, dbuf0, sem0)

        def zbody(r, _):
            for u in range(N // (16 * UNROLL)):
                for v in range(UNROLL):
                    cblk[pl.ds(r * N + (u * UNROLL + v) * 16, 16)] = zeros16
            return 0

        jax.lax.fori_loop(0, PASS_ROWS, zbody, 0)

        def process(sbuf, dbuf):
            def ibody(j, _):
                idxs = []
                msks = []
                for u in range(UNROLL):
                    sl = pl.ds(j * (16 * UNROLL) + u * 16, 16)
                    d = dbuf[sl]
                    s = sbuf[sl]
                    rel = d - base
                    msks.append(rel.astype(jnp.uint32) < jnp.uint32(PASS_ROWS))
                    idxs.append((rel << 11) + s)
                for u in range(UNROLL):
                    plsc.addupdate_scatter(cblk, [idxs[u]], ones16,
                                           mask=msks[u])
                return 0

            jax.lax.fori_loop(0, ECHUNK // (16 * UNROLL), ibody, 0)

        def cbody(i, _):
            c = i * 2
            start(c + 1, sbuf1, dbuf1, sem1)
            wait(sbuf0, dbuf0, sem0)
            process(sbuf0, dbuf0)

            @pl.when(c + 2 < NCHUNK)
            def _():
                start(c + 2, sbuf0, dbuf0, sem0)

            wait(sbuf1, dbuf1, sem1)
            process(sbuf1, dbuf1)
            return 0

        jax.lax.fori_loop(0, NCHUNK // 2, cbody, 0)
        pltpu.sync_copy(cblk, c_hbm.at[pl.ds(base * N, PASS_WORDS)])


@functools.lru_cache(maxsize=1)
def _sc_counts():
    return pl.kernel(
        _sc_counts_body,
        out_type=jax.ShapeDtypeStruct((N * N,), jnp.float32),
        mesh=plsc.VectorSubcoreMesh(
            core_axis_name="c", subcore_axis_name="s",
            num_cores=2, num_subcores=16,
        ),
        scratch_types=[
            pltpu.VMEM((PASS_WORDS,), jnp.float32),
            pltpu.VMEM((ECHUNK,), jnp.int32),
            pltpu.VMEM((ECHUNK,), jnp.int32),
            pltpu.VMEM((ECHUNK,), jnp.int32),
            pltpu.VMEM((ECHUNK,), jnp.int32),
            pltpu.SemaphoreType.DMA,
            pltpu.SemaphoreType.DMA,
        ],
        compiler_params=pltpu.CompilerParams(needs_layout_passes=False),
    )


def _b16(x):
    return x.astype(jnp.bfloat16)


def _r16(x):
    return x.astype(jnp.bfloat16).astype(jnp.float32)


def _mm(a, b):
    return jax.lax.dot_general(
        a, b, (((1,), (0,)), ((), ())),
        preferred_element_type=jnp.float32,
    )


def _mmT(a, b):
    return jax.lax.dot_general(
        a, b, (((1,), (1,)), ((), ())),
        preferred_element_type=jnp.float32,
    )


def _mmd(a, b):
    return _mm(_b16(a), _b16(b))


def _mmdT(a, b):
    return _mmT(_b16(a), _b16(b))


def _split16(x):
    hi = _b16(x)
    lo = _b16(x - hi.astype(jnp.float32))
    return hi, lo


def _mmx(a, b):
    ah, al = _split16(a)
    bh, bl = _split16(b)
    return _mm(ah, bh) + (_mm(ah, bl) + _mm(al, bh))


def _mmxT(a, b):
    ah, al = _split16(a)
    bh, bl = _split16(b)
    return _mmT(ah, bh) + (_mmT(ah, bl) + _mmT(al, bh))


def _lnv(x, g, b):
    m = jnp.mean(x, axis=-1, keepdims=True)
    v = jnp.mean((x - m) ** 2, axis=-1, keepdims=True)
    return (x - m) / jnp.sqrt(v + 1e-5) * g + b


def _relu(x):
    return jnp.maximum(x, 0.0)


def _tc_body(*refs):
    (coords, amask, speeds, dist, ttg, C,
     te_W, te_b, te_g, te_be,
     gte_W, gte_b, gte_g, gte_be,
     res0_W, res0_b,
     Wq0, bq0, Wk0, bk0, Wv0, bv0, Ws0, bs0,
     Wq1, bq1, Wk1, bk1, Wv1, bv1, Ws1, bs1,
     Wq2, bq2, Wk2, bk2, Wv2, bv2, Ws2, bs2,
     Wq3, bq3, Wk3, bk3, Wv3, bv3, Ws3, bs3,
     gff_W, gff_b, gff_g, gff_be,
     gout_W, gout_b,
     aWq, abq, aWk, abk, aWv, abv, aWo, abo,
     ff_W, ff_b, ff_g, ff_be,
     as_W, as_b, as_g, as_be,
     ao_W, ao_b,
     cr_W1, cr_b1, cr_g, cr_be,
     cr_W2, cr_b2,
     logits, values,
     cur_a, cur_b, q_s, k_s, v_s, sf_s, res_s, ctx_s, act_s) = refs

    tev = _lnv(
        _relu(_r16(ttg[...]) * _r16(te_W[0:1, :]) + te_b[...]),
        te_g[...], te_be[...],
    )
    combined = jnp.concatenate(
        [coords[...], amask[...], speeds[...], dist[...], ttg[...], tev], axis=1
    )
    tfeat = _lnv(
        _relu(_mmd(tev[:, 61:64], gte_W[...]) + gte_b[...]),
        gte_g[...], gte_be[...],
    )

    def gnn_layer(cur_val, res_ref, nxt_ref, Wq, bq, Wk, bk, Wv, bv, Ws, bs):
        q_s[...] = _mmd(cur_val, Wq[...]) + bq[...]
        k_s[...] = _mmd(cur_val, Wk[...]) + bk[...]
        v_s[...] = _mmd(cur_val, Wv[...]) + bv[...]
        sf_s[...] = _mmd(cur_val, Ws[...]) + bs[...]

        def blk(b, _):
            sl = pl.ds(b * BLK, BLK)
            L = _mmxT(q_s[sl, :], k_s[...]) * 0.125
            Cb = C[sl, :]
            msk = Cb > 0.0
            m = jnp.max(jnp.where(msk, L, -1e30), axis=1, keepdims=True)
            m = jnp.where(m < -9e29, 0.0, m)
            e = Cb * jnp.exp(jnp.minimum(L - m, 0.0))
            ssum = jnp.sum(e, axis=1, keepdims=True) + 1e-16
            agg = _mmx(e, v_s[...]) * (1.0 / ssum)
            nxt_ref[sl, :] = _relu(agg + sf_s[sl, :]) + res_ref[sl, :]
            return 0

        jax.lax.fori_loop(0, NBLK, blk, 0)

    res_s[...] = _mmd(combined, res0_W[...]) + res0_b[...]
    gnn_layer(combined, res_s, cur_b, Wq0, bq0, Wk0, bk0, Wv0, bv0, Ws0, bs0)
    gnn_layer(cur_b[...], cur_b, cur_a, Wq1, bq1, Wk1, bk1, Wv1, bv1, Ws1, bs1)
    gnn_layer(cur_a[...], cur_a, cur_b, Wq2, bq2, Wk2, bk2, Wv2, bv2, Ws2, bs2)
    gnn_layer(cur_b[...], cur_b, cur_a, Wq3, bq3, Wk3, bk3, Wv3, bv3, Ws3, bs3)

    comb2 = jnp.concatenate([cur_a[...], tfeat], axis=1)
    fusedg = _lnv(
        _relu(_mmd(comb2, gff_W[...]) + gff_b[...]), gff_g[...], gff_be[...]
    )
    gnn_out = _mmd(fusedg, gout_W[...]) + gout_b[...]

    q_s[...] = _mmd(tev, aWq[...]) + abq[...]
    k_s[...] = _mmd(tev, aWk[...]) + abk[...]
    v_s[...] = _mmd(tev, aWv[...]) + abv[...]

    def ablk(b, _):
        sl = pl.ds(b * BLK, BLK)
        S = _mmdT(q_s[sl, :], k_s[...]) * 0.125
        m = jnp.max(S, axis=1, keepdims=True)
        e = jnp.exp(S - m)
        ctx_s[sl, :] = _mmd(e * (1.0 / jnp.sum(e, axis=1, keepdims=True)), v_s[...])
        return 0

    jax.lax.fori_loop(0, NBLK, ablk, 0)
    tctx = _mmd(ctx_s[...], aWo[...]) + abo[...]

    fused = _lnv(
        _relu(
            _mmd(jnp.concatenate([gnn_out, tev, tctx], axis=1), ff_W[...])
            + ff_b[...]
        ),
        ff_g[...], ff_be[...],
    )
    act_s[...] = _lnv(
        _relu(_mmd(fused, as_W[...]) + as_b[...]), as_g[...], as_be[...]
    )

    def lblk(b, _):
        sl = pl.ds(b * BLK, BLK)
        logits[sl, :] = _mmd(act_s[sl, :], ao_W[...]) + ao_b[...]
        return 0

    jax.lax.fori_loop(0, NBLK, lblk, 0)

    h = _lnv(
        _relu(_mmd(fused, cr_W1[...]) + cr_b1[...]), cr_g[...], cr_be[...]
    )
    values[...] = _mmd(h, cr_W2[...]) + cr_b2[...]


_TC_SCRATCH = [
    pltpu.VMEM((N, 64), jnp.float32),
    pltpu.VMEM((N, 64), jnp.float32),
    pltpu.VMEM((N, 64), jnp.float32),
    pltpu.VMEM((N, 64), jnp.float32),
    pltpu.VMEM((N, 64), jnp.float32),
    pltpu.VMEM((N, 64), jnp.float32),
    pltpu.VMEM((N, 64), jnp.float32),
    pltpu.VMEM((N, 64), jnp.float32),
    pltpu.VMEM((N, 128), jnp.float32),
]

_TC_OUT = [
    jax.ShapeDtypeStruct((N, N), jnp.float32),
    jax.ShapeDtypeStruct((N, 1), jnp.float32),
]


def _tc_args(coords, amask_f, speeds, dist, ttg, C, p):
    col = lambda v: v.reshape(N, 1)
    row = lambda v: v.reshape(1, -1)
    args = [coords, col(amask_f), col(speeds), col(dist), col(ttg), C,
            p["te_W"], row(p["te_b"]), row(p["te_g"]), row(p["te_beta"]),
            p["gte_W"], row(p["gte_b"]), row(p["gte_g"]), row(p["gte_beta"]),
            p["res0_W"], row(p["res0_b"])]
    for l in range(4):
        args += [p["conv_Wq"][l], row(p["conv_bq"][l]),
                 p["conv_Wk"][l], row(p["conv_bk"][l]),
                 p["conv_Wv"][l], row(p["conv_bv"][l]),
                 p["conv_Ws"][l], row(p["conv_bs"][l])]
    args += [p["gff_W"], row(p["gff_b"]), row(p["gff_g"]), row(p["gff_beta"]),
             p["gout_W"], row(p["gout_b"]),
             p["att_Wq"], row(p["att_bq"]), p["att_Wk"], row(p["att_bk"]),
             p["att_Wv"], row(p["att_bv"]), p["att_Wo"], row(p["att_bo"]),
             p["ff_W"], row(p["ff_b"]), row(p["ff_g"]), row(p["ff_beta"]),
             p["as_W"], row(p["as_b"]), row(p["as_g"]), row(p["as_beta"]),
             p["ao_W"], row(p["ao_b"]),
             p["cr_W1"], row(p["cr_b1"]), row(p["cr_g"]), row(p["cr_beta"]),
             p["cr_W2"], row(p["cr_b2"])]
    return args


_tc_forward = pl.pallas_call(
    _tc_body,
    out_shape=_TC_OUT,
    scratch_shapes=_TC_SCRATCH,
    compiler_params=pltpu.CompilerParams(
        vmem_limit_bytes=100 * 1024 * 1024
    ),
)


def kernel(mission_coords, edge_index, batch, uavs_info, action_mask,
           speeds, dist_matrix, timetogo_matrix, params):
    src = edge_index[0]
    dst = edge_index[1]
    C = _sc_counts()(src, dst).reshape(N, N)
    args = _tc_args(
        mission_coords, action_mask.astype(jnp.float32), speeds,
        dist_matrix, timetogo_matrix, C, params,
    )
    logits, values = _tc_forward(*args)
    return logits, values[:, 0]


# speedup vs baseline: 2.0981x; 1.0005x over previous
import functools

import jax
import jax.numpy as jnp
from jax.experimental import pallas as pl
from jax.experimental.pallas import tpu as pltpu
from jax.experimental.pallas import tpu_sc as plsc

N = 2048
E = 131072
NW = 32
ROWS_PER_W = N // NW
PASS_ROWS = ROWS_PER_W // 2
PASS_WORDS = PASS_ROWS * N
ECHUNK = 8192
BLK = 256
NBLK = N // BLK


NCHUNK = E // ECHUNK
UNROLL = 16


def _sc_counts_body(src_hbm, dst_hbm, c_hbm,
                    cblk, sbuf0, dbuf0, sbuf1, dbuf1, sem0, sem1):
    wid = jax.lax.axis_index("s") * 2 + jax.lax.axis_index("c")
    zeros16 = jnp.zeros((16,), jnp.float32)
    ones16 = jnp.ones((16,), jnp.float32)

    def start(c, sbuf, dbuf, sem):
        off = c * ECHUNK
        pltpu.make_async_copy(src_hbm.at[pl.ds(off, ECHUNK)], sbuf, sem).start()
        pltpu.make_async_copy(dst_hbm.at[pl.ds(off, ECHUNK)], dbuf, sem).start()

    def wait(sbuf, dbuf, sem):
        pltpu.make_async_copy(src_hbm.at[pl.ds(0, ECHUNK)], sbuf, sem).wait()
        pltpu.make_async_copy(dst_hbm.at[pl.ds(0, ECHUNK)], dbuf, sem).wait()

    for p in range(2):
        base = wid * ROWS_PER_W + p * PASS_ROWS
        start(0, sbuf0, dbuf0, sem0)

        def zbody(r, _):
            for u in range(N // (16 * UNROLL)):
                for v in range(UNROLL):
                    cblk[pl.ds(r * N + (u * UNROLL + v) * 16, 16)] = zeros16
            return 0

        jax.lax.fori_loop(0, PASS_ROWS, zbody, 0)

        def process(sbuf, dbuf):
            def ibody(j, _):
                idxs = []
                msks = []
                for u in range(UNROLL):
                    sl = pl.ds(j * (16 * UNROLL) + u * 16, 16)
                    d = dbuf[sl]
                    s = sbuf[sl]
                    rel = d - base
                    msks.append(rel.astype(jnp.uint32) < jnp.uint32(PASS_ROWS))
                    idxs.append((rel << 11) + s)
                for u in range(UNROLL):
                    plsc.addupdate_scatter(cblk, [idxs[u]], ones16,
                                           mask=msks[u])
                return 0

            jax.lax.fori_loop(0, ECHUNK // (16 * UNROLL), ibody, 0)

        def cbody(i, _):
            c = i * 2
            start(c + 1, sbuf1, dbuf1, sem1)
            wait(sbuf0, dbuf0, sem0)
            process(sbuf0, dbuf0)

            @pl.when(c + 2 < NCHUNK)
            def _():
                start(c + 2, sbuf0, dbuf0, sem0)

            wait(sbuf1, dbuf1, sem1)
            process(sbuf1, dbuf1)
            return 0

        jax.lax.fori_loop(0, NCHUNK // 2, cbody, 0)
        pltpu.sync_copy(cblk, c_hbm.at[pl.ds(base * N, PASS_WORDS)])


@functools.lru_cache(maxsize=1)
def _sc_counts():
    return pl.kernel(
        _sc_counts_body,
        out_type=jax.ShapeDtypeStruct((N * N,), jnp.float32),
        mesh=plsc.VectorSubcoreMesh(
            core_axis_name="c", subcore_axis_name="s",
            num_cores=2, num_subcores=16,
        ),
        scratch_types=[
            pltpu.VMEM((PASS_WORDS,), jnp.float32),
            pltpu.VMEM((ECHUNK,), jnp.int32),
            pltpu.VMEM((ECHUNK,), jnp.int32),
            pltpu.VMEM((ECHUNK,), jnp.int32),
            pltpu.VMEM((ECHUNK,), jnp.int32),
            pltpu.SemaphoreType.DMA,
            pltpu.SemaphoreType.DMA,
        ],
        compiler_params=pltpu.CompilerParams(needs_layout_passes=False),
    )


def _b16(x):
    return x.astype(jnp.bfloat16)


def _r16(x):
    return x.astype(jnp.bfloat16).astype(jnp.float32)


def _mm(a, b):
    return jax.lax.dot_general(
        a, b, (((1,), (0,)), ((), ())),
        preferred_element_type=jnp.float32,
    )


def _mmT(a, b):
    return jax.lax.dot_general(
        a, b, (((1,), (1,)), ((), ())),
        preferred_element_type=jnp.float32,
    )


def _mmd(a, b):
    return _mm(_b16(a), _b16(b))


def _mmdT(a, b):
    return _mmT(_b16(a), _b16(b))


def _split16(x):
    hi = _b16(x)
    lo = _b16(x - hi.astype(jnp.float32))
    return hi, lo


def _mmx(a, b):
    ah, al = _split16(a)
    bh, bl = _split16(b)
    return _mm(ah, bh) + (_mm(ah, bl) + _mm(al, bh))


def _mmxT(a, b):
    ah, al = _split16(a)
    bh, bl = _split16(b)
    return _mmT(ah, bh) + (_mmT(ah, bl) + _mmT(al, bh))


def _lnv(x, g, b):
    m = jnp.mean(x, axis=-1, keepdims=True)
    v = jnp.mean((x - m) ** 2, axis=-1, keepdims=True)
    return (x - m) / jnp.sqrt(v + 1e-5) * g + b


def _relu(x):
    return jnp.maximum(x, 0.0)


def _tc_body(*refs):
    (coords, amask, speeds, dist, ttg, C,
     te_W, te_b, te_g, te_be,
     gte_W, gte_b, gte_g, gte_be,
     res0_W, res0_b,
     Wq0, bq0, Wk0, bk0, Wv0, bv0, Ws0, bs0,
     Wq1, bq1, Wk1, bk1, Wv1, bv1, Ws1, bs1,
     Wq2, bq2, Wk2, bk2, Wv2, bv2, Ws2, bs2,
     Wq3, bq3, Wk3, bk3, Wv3, bv3, Ws3, bs3,
     gff_W, gff_b, gff_g, gff_be,
     gout_W, gout_b,
     aWq, abq, aWk, abk, aWv, abv, aWo, abo,
     ff_W, ff_b, ff_g, ff_be,
     as_W, as_b, as_g, as_be,
     ao_W, ao_b,
     cr_W1, cr_b1, cr_g, cr_be,
     cr_W2, cr_b2,
     logits, values,
     cur_a, cur_b, q_s, k_s, v_s, sf_s, res_s, ctx_s, act_s) = refs

    tev = _lnv(
        _relu(_r16(ttg[...]) * _r16(te_W[0:1, :]) + te_b[...]),
        te_g[...], te_be[...],
    )
    combined = jnp.concatenate(
        [coords[...], amask[...], speeds[...], dist[...], ttg[...], tev], axis=1
    )
    tfeat = _lnv(
        _relu(_mmd(tev[:, 61:64], gte_W[...]) + gte_b[...]),
        gte_g[...], gte_be[...],
    )

    def gnn_layer(cur_val, res_ref, nxt_ref, Wq, bq, Wk, bk, Wv, bv, Ws, bs):
        q_s[...] = _mmd(cur_val, Wq[...]) + bq[...]
        k_s[...] = _mmd(cur_val, Wk[...]) + bk[...]
        v_s[...] = _mmd(cur_val, Wv[...]) + bv[...]
        sf_s[...] = _mmd(cur_val, Ws[...]) + bs[...]

        def blk(b, _):
            sl = pl.ds(b * BLK, BLK)
            L = _mmxT(q_s[sl, :], k_s[...]) * 0.125
            Cb = C[sl, :]
            msk = Cb > 0.0
            m = jnp.max(jnp.where(msk, L, -1e30), axis=1, keepdims=True)
            m = jnp.where(m < -9e29, 0.0, m)
            e = Cb * jnp.exp(jnp.minimum(L - m, 0.0))
            ssum = jnp.sum(e, axis=1, keepdims=True) + 1e-16
            agg = _mmx(e, v_s[...]) * (1.0 / ssum)
            nxt_ref[sl, :] = _relu(agg + sf_s[sl, :]) + res_ref[sl, :]
            return 0

        jax.lax.fori_loop(0, NBLK, blk, 0)

    res_s[...] = _mmd(combined, res0_W[...]) + res0_b[...]
    gnn_layer(combined, res_s, cur_b, Wq0, bq0, Wk0, bk0, Wv0, bv0, Ws0, bs0)
    gnn_layer(cur_b[...], cur_b, cur_a, Wq1, bq1, Wk1, bk1, Wv1, bv1, Ws1, bs1)
    gnn_layer(cur_a[...], cur_a, cur_b, Wq2, bq2, Wk2, bk2, Wv2, bv2, Ws2, bs2)
    gnn_layer(cur_b[...], cur_b, cur_a, Wq3, bq3, Wk3, bk3, Wv3, bv3, Ws3, bs3)

    comb2 = jnp.concatenate([cur_a[...], tfeat], axis=1)
    fusedg = _lnv(
        _relu(_mmd(comb2, gff_W[...]) + gff_b[...]), gff_g[...], gff_be[...]
    )
    gnn_out = _mmd(fusedg, gout_W[...]) + gout_b[...]

    q_s[...] = _mmd(tev, aWq[...]) + abq[...]
    k_s[...] = _mmd(tev, aWk[...]) + abk[...]
    v_s[...] = _mmd(tev, aWv[...]) + abv[...]

    def ablk(b, _):
        sl = pl.ds(b * BLK, BLK)
        S = _mmdT(q_s[sl, :], k_s[...]) * 0.125
        m = jnp.max(S, axis=1, keepdims=True)
        e = jnp.exp(S - m)
        ctx_s[sl, :] = _mmd(e * (1.0 / jnp.sum(e, axis=1, keepdims=True)), v_s[...])
        return 0

    jax.lax.fori_loop(0, NBLK, ablk, 0)
    tctx = _mmd(ctx_s[...], aWo[...]) + abo[...]

    fused = _lnv(
        _relu(
            _mmd(jnp.concatenate([gnn_out, tev, tctx], axis=1), ff_W[...])
            + ff_b[...]
        ),
        ff_g[...], ff_be[...],
    )
    act_s[...] = _lnv(
        _relu(_mmd(fused, as_W[...]) + as_b[...]), as_g[...], as_be[...]
    )

    def lblk(b, _):
        sl = pl.ds(b * BLK, BLK)
        logits[sl, :] = _mmd(act_s[sl, :], ao_W[...]) + ao_b[...]
        return 0

    jax.lax.fori_loop(0, NBLK, lblk, 0)

    h = _lnv(
        _relu(_mmd(fused, cr_W1[...]) + cr_b1[...]), cr_g[...], cr_be[...]
    )
    values[...] = _mmd(h, cr_W2[...]) + cr_b2[...]


_TC_SCRATCH = [
    pltpu.VMEM((N, 64), jnp.float32),
    pltpu.VMEM((N, 64), jnp.float32),
    pltpu.VMEM((N, 64), jnp.float32),
    pltpu.VMEM((N, 64), jnp.float32),
    pltpu.VMEM((N, 64), jnp.float32),
    pltpu.VMEM((N, 64), jnp.float32),
    pltpu.VMEM((N, 64), jnp.float32),
    pltpu.VMEM((N, 64), jnp.float32),
    pltpu.VMEM((N, 128), jnp.float32),
]

_TC_OUT = [
    jax.ShapeDtypeStruct((N, N), jnp.float32),
    jax.ShapeDtypeStruct((N, 1), jnp.float32),
]


def _tc_args(coords, amask_f, speeds, dist, ttg, C, p):
    col = lambda v: v.reshape(N, 1)
    row = lambda v: v.reshape(1, -1)
    args = [coords, col(amask_f), col(speeds), col(dist), col(ttg), C,
            p["te_W"], row(p["te_b"]), row(p["te_g"]), row(p["te_beta"]),
            p["gte_W"], row(p["gte_b"]), row(p["gte_g"]), row(p["gte_beta"]),
            p["res0_W"], row(p["res0_b"])]
    for l in range(4):
        args += [p["conv_Wq"][l], row(p["conv_bq"][l]),
                 p["conv_Wk"][l], row(p["conv_bk"][l]),
                 p["conv_Wv"][l], row(p["conv_bv"][l]),
                 p["conv_Ws"][l], row(p["conv_bs"][l])]
    args += [p["gff_W"], row(p["gff_b"]), row(p["gff_g"]), row(p["gff_beta"]),
             p["gout_W"], row(p["gout_b"]),
             p["att_Wq"], row(p["att_bq"]), p["att_Wk"], row(p["att_bk"]),
             p["att_Wv"], row(p["att_bv"]), p["att_Wo"], row(p["att_bo"]),
             p["ff_W"], row(p["ff_b"]), row(p["ff_g"]), row(p["ff_beta"]),
             p["as_W"], row(p["as_b"]), row(p["as_g"]), row(p["as_beta"]),
             p["ao_W"], row(p["ao_b"]),
             p["cr_W1"], row(p["cr_b1"]), row(p["cr_g"]), row(p["cr_beta"]),
             p["cr_W2"], row(p["cr_b2"])]
    return args


_tc_forward = pl.pallas_call(
    _tc_body,
    out_shape=_TC_OUT,
    scratch_shapes=_TC_SCRATCH,
    compiler_params=pltpu.CompilerParams(
        vmem_limit_bytes=100 * 1024 * 1024
    ),
)


def kernel(mission_coords, edge_index, batch, uavs_info, action_mask,
           speeds, dist_matrix, timetogo_matrix, params):
    src = edge_index[0]
    dst = edge_index[1]
    C = _sc_counts()(src, dst).reshape(N, N)
    args = _tc_args(
        mission_coords, action_mask.astype(jnp.float32), speeds,
        dist_matrix, timetogo_matrix, C, params,
    )
    logits, values = _tc_forward(*args)
    return logits, values[:, 0]
